# Initial kernel scaffold; baseline (speedup 1.0000x reference)
import jax
import jax.numpy as jnp
from jax.experimental import pallas as pl


def kernel(sr_data, tg_data, W_sr, W_tg):
    raise NotImplementedError("write your pallas kernel here")


# speedup vs baseline: 1.6311x; 1.6311x over previous
import functools

import jax
import jax.numpy as jnp
from jax import lax
from jax.experimental import pallas as pl
from jax.experimental.pallas import tpu as pltpu
from jax.experimental.pallas import tpu_sc as plsc

NUM_ROWS = 16384
NUM_COLS = 20
EMBED_DIM = 32
B = NUM_ROWS * NUM_COLS

NC = 2
NS = 16
NW = NC * NS
B_PER_W = B // NW
CHUNK = 2048
N_CHUNKS = B_PER_W // CHUNK


@functools.partial(
    pl.kernel,
    mesh=plsc.VectorSubcoreMesh(core_axis_name="c", subcore_axis_name="s"),
    out_type=(
        jax.ShapeDtypeStruct((B, EMBED_DIM), jnp.float32),
        jax.ShapeDtypeStruct((B, EMBED_DIM), jnp.float32),
    ),
    scratch_types=[
        pltpu.VMEM((CHUNK,), jnp.int32),
        pltpu.VMEM((CHUNK, EMBED_DIM), jnp.float32),
        pltpu.SemaphoreType.DMA,
    ],
    compiler_params=pltpu.CompilerParams(use_tc_tiling_on_sc=False),
)
def _double_gather(w_sr, w_tg, idx_sr, idx_tg, o_sr, o_tg, idx_v, rows_v, sem):
    wid = lax.axis_index("s") * NC + lax.axis_index("c")
    base = wid * B_PER_W
    for w, idx, o in ((w_sr, idx_sr, o_sr), (w_tg, idx_tg, o_tg)):
        for g in range(N_CHUNKS):
            off = base + g * CHUNK
            pltpu.sync_copy(idx.at[pl.ds(off, CHUNK)], idx_v)
            pltpu.async_copy(w.at[idx_v], rows_v, sem).wait()
            pltpu.sync_copy(rows_v, o.at[pl.ds(off, CHUNK)])


def kernel(sr_data, tg_data, W_sr, W_tg):
    idx_sr = sr_data.reshape(B)
    idx_tg = tg_data.reshape(B)
    o_sr, o_tg = _double_gather(W_sr, W_tg, idx_sr, idx_tg)
    return (
        o_sr.reshape(NUM_ROWS, NUM_COLS, EMBED_DIM),
        o_tg.reshape(NUM_ROWS, NUM_COLS, EMBED_DIM),
    )


# speedup vs baseline: 1.6412x; 1.0062x over previous
import functools

import jax
import jax.numpy as jnp
from jax import lax
from jax.experimental import pallas as pl
from jax.experimental.pallas import tpu as pltpu
from jax.experimental.pallas import tpu_sc as plsc

NUM_ROWS = 16384
NUM_COLS = 20
EMBED_DIM = 32
B = NUM_ROWS * NUM_COLS

NC = 2
NS = 16
NW = NC * NS
B_PER_W = B // NW
CHUNK = 1024
N_CHUNKS = B_PER_W // CHUNK
NBUF = 3


@functools.partial(
    pl.kernel,
    mesh=plsc.VectorSubcoreMesh(core_axis_name="c", subcore_axis_name="s"),
    out_type=(
        jax.ShapeDtypeStruct((B, EMBED_DIM), jnp.float32),
        jax.ShapeDtypeStruct((B, EMBED_DIM), jnp.float32),
    ),
    scratch_types=[
        pltpu.VMEM((2, B_PER_W), jnp.int32),
        pltpu.VMEM((NBUF, CHUNK, EMBED_DIM), jnp.float32),
        pltpu.SemaphoreType.DMA((2,)),
        pltpu.SemaphoreType.DMA((NBUF,)),
        pltpu.SemaphoreType.DMA((NBUF,)),
    ],
    compiler_params=pltpu.CompilerParams(use_tc_tiling_on_sc=False),
)
def _double_gather(w_sr, w_tg, idx_sr, idx_tg, o_sr, o_tg,
                   idx_v, rows_v, isem, gsem, wsem):
    wid = lax.axis_index("s") * NC + lax.axis_index("c")
    base = wid * B_PER_W
    icopy = [
        pltpu.async_copy(idx_sr.at[pl.ds(base, B_PER_W)], idx_v.at[0], isem.at[0]),
        pltpu.async_copy(idx_tg.at[pl.ds(base, B_PER_W)], idx_v.at[1], isem.at[1]),
    ]
    for t, (w, o) in enumerate(((w_sr, o_sr), (w_tg, o_tg))):
        icopy[t].wait()
        gcopy = [None] * N_CHUNKS
        wcopy = [None] * N_CHUNKS
        for g in range(N_CHUNKS):
            b = g % NBUF
            if g >= NBUF:
                wcopy[g - NBUF].wait()
            gcopy[g] = pltpu.async_copy(
                w.at[idx_v.at[t].at[pl.ds(g * CHUNK, CHUNK)]],
                rows_v.at[b], gsem.at[b])
            if g >= 1:
                p = g - 1
                gcopy[p].wait()
                wcopy[p] = pltpu.async_copy(
                    rows_v.at[p % NBUF],
                    o.at[pl.ds(base + p * CHUNK, CHUNK)], wsem.at[p % NBUF])
        p = N_CHUNKS - 1
        gcopy[p].wait()
        wcopy[p] = pltpu.async_copy(
            rows_v.at[p % NBUF],
            o.at[pl.ds(base + p * CHUNK, CHUNK)], wsem.at[p % NBUF])
        for p in range(N_CHUNKS - NBUF, N_CHUNKS):
            wcopy[p].wait()


def kernel(sr_data, tg_data, W_sr, W_tg):
    idx_sr = sr_data.reshape(B)
    idx_tg = tg_data.reshape(B)
    o_sr, o_tg = _double_gather(W_sr, W_tg, idx_sr, idx_tg)
    return (
        o_sr.reshape(NUM_ROWS, NUM_COLS, EMBED_DIM),
        o_tg.reshape(NUM_ROWS, NUM_COLS, EMBED_DIM),
    )
